# Initial kernel scaffold; baseline (speedup 1.0000x reference)
import jax
import jax.numpy as jnp
from jax.experimental import pallas as pl


def kernel(constraint_features, edge_index, edge_attr, variable_features, batch, params):
    raise NotImplementedError("write your pallas kernel here")


# speedup vs baseline: 20.4152x; 20.4152x over previous
import functools

import jax
import jax.numpy as jnp
from jax import lax
from jax.experimental import pallas as pl
from jax.experimental.pallas import tpu as pltpu
from jax.experimental.pallas import tpu_sc as plsc

F32 = jnp.float32
EPS = 1e-5
NUM_GRAPHS = 128
KTOP = 16
NCONV = 4
NSC = 2
NSUB = 16


def _ln_rows(x, g, b):
    m = jnp.mean(x, axis=-1, keepdims=True)
    xc = x - m
    v = jnp.mean(xc * xc, axis=-1, keepdims=True)
    return xc * lax.rsqrt(v + EPS) * g + b


def _var_table_body(x_ref, g12_ref, b12_ref, w1t_ref, b1_ref, w2t_ref, b2_ref,
                    wrt_ref, base_ref, fg_ref, fb_ref, wft_ref, bf_ref, out_ref):
    x = _ln_rows(x_ref[...], g12_ref[...], b12_ref[...])
    h = jax.nn.relu(jnp.dot(x, w1t_ref[...], preferred_element_type=F32) + b1_ref[...])
    h = jax.nn.relu(jnp.dot(h, w2t_ref[...], preferred_element_type=F32) + b2_ref[...])
    for c in range(NCONV):
        t = jnp.dot(h, wrt_ref[c], preferred_element_type=F32) + base_ref[c]
        ln = _ln_rows(t, fg_ref[c], fb_ref[c])
        u = jnp.dot(jax.nn.relu(ln), wft_ref[c], preferred_element_type=F32) + bf_ref[c]
        out_ref[c] = u


def _post_body(a_ref, pg_ref, pb_ref, at_ref, c2_ref, wo2t_ref, bo2_ref, out_ref):
    for c in range(NCONV):
        ln = _ln_rows(a_ref[c], pg_ref[c], pb_ref[c])
        z = jax.nn.relu(jnp.dot(ln, at_ref[c], preferred_element_type=F32) + c2_ref[c])
        y = jnp.dot(z, wo2t_ref[c], preferred_element_type=F32) + bo2_ref[c]
        out_ref[c] = y


def _mlp_body(x_ref, w1t_ref, b1_ref, w2t_ref, b2_ref, o_ref):
    h = jax.nn.relu(jnp.dot(x_ref[...], w1t_ref[...], preferred_element_type=F32)
                    + b1_ref[...])
    y = jnp.dot(h, w2t_ref[...], preferred_element_type=F32) + b2_ref[...]
    o_ref[...] = 1.0 / (1.0 + jnp.exp(-y))


def _full_spec(shape):
    return pl.BlockSpec(shape, lambda i: tuple(0 for _ in shape))


def kernel(constraint_features, edge_index, edge_attr, variable_features, batch, params):
    p = params
    nvars, fvar = variable_features.shape
    ncons = constraint_features.shape[0]
    nedges = edge_index.shape[1]
    del constraint_features, edge_attr

    ef = p['ee_ln_b'][0]
    cpre = p['ce_ln_b'][0]
    x1 = jax.nn.relu(cpre * p['ce_W1'][:, 0] + p['ce_b1'])
    cv = jax.nn.relu(p['ce_W2'] @ x1 + p['ce_b2'])

    convs = p['convs']
    stk = lambda f: jnp.stack([f(c) for c in convs])
    wrt = stk(lambda c: c['Wr'].T)
    base = stk(lambda c: (c['Wl'] @ cv + c['bl'] + ef * c['We'][:, 0])[None, :])
    fg = stk(lambda c: c['fin_ln_g'][None, :])
    fb = stk(lambda c: c['fin_ln_b'][None, :])
    wft = stk(lambda c: c['Wf'].T)
    bf = stk(lambda c: c['bf'][None, :])
    pg = stk(lambda c: c['post_ln_g'][None, :])
    pb = stk(lambda c: c['post_ln_b'][None, :])
    at_ = stk(lambda c: c['Wo1'][:, :16].T)
    c2 = stk(lambda c: (c['Wo1'][:, 16:] @ cv + c['bo1'])[None, :])
    wo2t = stk(lambda c: c['Wo2'].T)
    bo2 = stk(lambda c: c['bo2'][None, :])

    RA = 1000
    u4 = pl.pallas_call(
        _var_table_body,
        grid=(nvars // RA,),
        in_specs=[pl.BlockSpec((RA, fvar), lambda i: (i, 0)),
                  _full_spec((1, fvar)), _full_spec((1, fvar)),
                  _full_spec((fvar, 16)), _full_spec((1, 16)),
                  _full_spec((16, 16)), _full_spec((1, 16)),
                  _full_spec((NCONV, 16, 16)), _full_spec((NCONV, 1, 16)),
                  _full_spec((NCONV, 1, 16)), _full_spec((NCONV, 1, 16)),
                  _full_spec((NCONV, 16, 16)), _full_spec((NCONV, 1, 16))],
        out_specs=pl.BlockSpec((NCONV, RA, 16), lambda i: (0, i, 0)),
        out_shape=jax.ShapeDtypeStruct((NCONV, nvars, 16), F32),
    )(variable_features,
      p['ve_ln_g'][None, :], p['ve_ln_b'][None, :],
      p['ve_W1'].T, p['ve_b1'][None, :], p['ve_W2'].T, p['ve_b2'][None, :],
      wrt, base, fg, fb, wft, bf)

    ncons_p = 100096
    stripe = ncons_p // NSUB
    BLK = 4
    cpw = -(-nedges // (NSUB * 128))
    cps = -(-cpw // BLK) * BLK
    nblk = cps // BLK
    epad = NSUB * cps * 128
    srcp = jnp.concatenate(
        [edge_index[0], jnp.zeros((epad - nedges,), jnp.int32)]
    ).reshape(NSUB, nblk, BLK, 128)
    dstp = jnp.concatenate(
        [edge_index[1], jnp.full((epad - nedges,), ncons, jnp.int32)]
    ).reshape(NSUB, nblk, BLK, 128)
    zeros_hbm = jnp.zeros((stripe, 16), F32)

    mesh = plsc.VectorSubcoreMesh(core_axis_name="c", subcore_axis_name="s",
                                  num_cores=NSC, num_subcores=NSUB)

    @functools.partial(
        pl.kernel, mesh=mesh,
        compiler_params=pltpu.CompilerParams(use_tc_tiling_on_sc=False, needs_layout_passes=False),
        out_type=jax.ShapeDtypeStruct((NCONV, ncons_p, 16), F32),
        scratch_types=[
            pltpu.VMEM((BLK, 128), jnp.int32),
            pltpu.VMEM((BLK, 128), jnp.int32),
            pltpu.VMEM((BLK, 128, 16), F32),
            pltpu.VMEM_SHARED((ncons_p, 16), F32),
            pltpu.SemaphoreType.DMA,
            pltpu.SemaphoreType.DMA,
        ])
    def _edge_kernel(u_hbm, src_hbm, dst_hbm, z_hbm, out_hbm,
                     sblk, dblk, rows, acc, gsem, ssem):
        core = lax.axis_index("c")
        sid = lax.axis_index("s")
        for pss in range(2):
            conv = core * 2 + pss
            pltpu.sync_copy(z_hbm, acc.at[pl.ds(sid * stripe, stripe)])
            plsc.subcore_barrier()
            utab = u_hbm.at[conv]

            def blk_body(b, _):
                pltpu.sync_copy(src_hbm.at[sid, b], sblk)
                pltpu.sync_copy(dst_hbm.at[sid, b], dblk)
                for j in range(BLK):
                    pltpu.async_copy(utab.at[sblk.at[j]], rows.at[j], gsem)
                for j in range(BLK):
                    pltpu.make_async_copy(utab.at[sblk.at[j]], rows.at[j], gsem).wait()
                for j in range(BLK):
                    pltpu.async_copy(rows.at[j], acc.at[dblk.at[j]], ssem, add=True)
                for j in range(BLK):
                    pltpu.make_async_copy(rows.at[j], acc.at[dblk.at[j]], ssem).wait()
                return 0

            lax.fori_loop(0, nblk, blk_body, 0)
            plsc.subcore_barrier()
            pltpu.sync_copy(acc.at[pl.ds(sid * stripe, stripe)],
                            out_hbm.at[conv].at[pl.ds(sid * stripe, stripe)])
            plsc.subcore_barrier()

    aggr4 = _edge_kernel(u4, srcp, dstp, zeros_hbm)

    RC = 800
    nh = 100128
    h4 = pl.pallas_call(
        _post_body,
        grid=(ncons // RC,),
        in_specs=[pl.BlockSpec((NCONV, RC, 16), lambda i: (0, i, 0)),
                  _full_spec((NCONV, 1, 16)), _full_spec((NCONV, 1, 16)),
                  _full_spec((NCONV, 16, 16)), _full_spec((NCONV, 1, 16)),
                  _full_spec((NCONV, 16, 16)), _full_spec((NCONV, 1, 16))],
        out_specs=pl.BlockSpec((NCONV, RC, 16), lambda i: (0, i, 0)),
        out_shape=jax.ShapeDtypeStruct((NCONV, nh, 16), F32),
    )(aggr4, pg, pb, at_, c2, wo2t, bo2)
    hrows = h4.reshape(NCONV * nh, 16)

    gr = jnp.arange(NUM_GRAPHS, dtype=batch.dtype)
    starts = jnp.searchsorted(batch, gr, side='left').astype(jnp.int32)
    ends = jnp.searchsorted(batch, gr, side='right').astype(jnp.int32)
    item_s = jnp.repeat(starts, NCONV)
    item_e = jnp.repeat(ends, NCONV)

    nitems = NUM_GRAPHS * NCONV
    per_sub = nitems // (NSC * NSUB)

    @functools.partial(
        pl.kernel, mesh=mesh,
        compiler_params=pltpu.CompilerParams(use_tc_tiling_on_sc=False, needs_layout_passes=False),
        out_type=jax.ShapeDtypeStruct((nitems * KTOP, 16), F32),
        scratch_types=[
            pltpu.VMEM((128, 16), F32),
            pltpu.VMEM((per_sub,), jnp.int32),
            pltpu.VMEM((per_sub,), jnp.int32),
            pltpu.VMEM((16,), jnp.int32),
            pltpu.VMEM((16, 16), F32),
            pltpu.SemaphoreType.DMA,
        ])
    def _pool_kernel(hr_hbm, its_hbm, ite_hbm, out_hbm,
                     stage, msv, mev, idxb, rows16, sem):
        core = lax.axis_index("c")
        sid = lax.axis_index("s")
        wid = sid * NSC + core
        pltpu.sync_copy(its_hbm.at[pl.ds(wid * per_sub, per_sub)], msv)
        pltpu.sync_copy(ite_hbm.at[pl.ds(wid * per_sub, per_sub)], mev)
        iot = lax.iota(jnp.int32, 16)
        NEG = jnp.float32(-jnp.inf)
        SENT = jnp.int32(2**31 - 1)
        lane15 = jnp.full((16,), 15, jnp.int32)

        def item_body(i, _):
            it = wid * per_sub + i
            cix = lax.rem(it, NCONV)
            sel = iot == i
            s = jnp.sum(jnp.where(sel, msv[...], 0))
            e = jnp.sum(jnp.where(sel, mev[...], 0))
            base0 = cix * nh
            s8 = jnp.bitwise_and(s, jnp.int32(-8))
            nst = (e - s8 + 127) // 128

            def stage_body(t, carry):
                runk, runi = carry
                r0 = base0 + s8 + t * 128
                pltpu.sync_copy(hr_hbm.at[pl.ds(r0, 128)], stage)
                for q in range(8):
                    kq = plsc.load_gather(stage, [q * 16 + iot, lane15])
                    ids = s8 + t * 128 + q * 16 + iot
                    valid = (ids >= s) & (ids < e)
                    kq = jnp.where(valid, kq, NEG)
                    idq = jnp.where(valid, ids, SENT)
                    sk, si = plsc.sort_key_val(kq, idq, descending=True)
                    rb = lax.rev(sk, (0,))
                    ri = lax.rev(si, (0,))
                    take = (runk > rb) | ((runk == rb) & (runi <= ri))
                    mk = jnp.where(take, runk, rb)
                    mi = jnp.where(take, runi, ri)
                    runk, runi = plsc.sort_key_val(mk, mi, descending=True)
                return runk, runi

            runk, runi = lax.fori_loop(
                0, nst, stage_body,
                (jnp.full((16,), NEG), jnp.full((16,), SENT)))
            valid16 = runi != SENT
            idxb[...] = base0 + jnp.where(valid16, runi, 0)
            pltpu.async_copy(hr_hbm.at[idxb], rows16, sem).wait()
            for r in range(16):
                vr = jnp.sum(jnp.where((iot == r) & valid16, 1, 0)) > 0
                rows16[r, :] = jnp.where(vr, rows16[r, :], jnp.zeros((16,), F32))
            pltpu.sync_copy(rows16, out_hbm.at[pl.ds(it * KTOP, KTOP)])
            return 0

        lax.fori_loop(0, per_sub, item_body, 0)

    pooled = _pool_kernel(hrows, item_s, item_e)

    cat = pooled.reshape(NUM_GRAPHS, NCONV * KTOP * 16)
    out = pl.pallas_call(
        _mlp_body,
        out_shape=jax.ShapeDtypeStruct((NUM_GRAPHS, 1), F32),
    )(cat, p['m_W1'].T, p['m_b1'][None, :], p['m_W2'].T, p['m_b2'][None, :])
    return out[:, 0]


# speedup vs baseline: 31.0631x; 1.5216x over previous
import functools

import jax
import jax.numpy as jnp
from jax import lax
from jax.experimental import pallas as pl
from jax.experimental.pallas import tpu as pltpu
from jax.experimental.pallas import tpu_sc as plsc

F32 = jnp.float32
EPS = 1e-5
NUM_GRAPHS = 128
KTOP = 16
NCONV = 4
NSC = 2
NSUB = 16


def _ln_rows(x, g, b):
    m = jnp.mean(x, axis=-1, keepdims=True)
    xc = x - m
    v = jnp.mean(xc * xc, axis=-1, keepdims=True)
    return xc * lax.rsqrt(v + EPS) * g + b


def _var_table_body(x_ref, g12_ref, b12_ref, w1t_ref, b1_ref, w2t_ref, b2_ref,
                    wrt_ref, base_ref, fg_ref, fb_ref, wft_ref, bf_ref, out_ref):
    x = _ln_rows(x_ref[...], g12_ref[...], b12_ref[...])
    h = jax.nn.relu(jnp.dot(x, w1t_ref[...], preferred_element_type=F32) + b1_ref[...])
    h = jax.nn.relu(jnp.dot(h, w2t_ref[...], preferred_element_type=F32) + b2_ref[...])
    for c in range(NCONV):
        t = jnp.dot(h, wrt_ref[c], preferred_element_type=F32) + base_ref[c]
        ln = _ln_rows(t, fg_ref[c], fb_ref[c])
        u = jnp.dot(jax.nn.relu(ln), wft_ref[c], preferred_element_type=F32) + bf_ref[c]
        out_ref[c] = u


def _post_body(a_ref, pg_ref, pb_ref, at_ref, c2_ref, wo2t_ref, bo2_ref, out_ref):
    for c in range(NCONV):
        ln = _ln_rows(a_ref[c], pg_ref[c], pb_ref[c])
        z = jax.nn.relu(jnp.dot(ln, at_ref[c], preferred_element_type=F32) + c2_ref[c])
        y = jnp.dot(z, wo2t_ref[c], preferred_element_type=F32) + bo2_ref[c]
        out_ref[c] = y


def _mlp_body(x_ref, w1t_ref, b1_ref, w2t_ref, b2_ref, o_ref):
    h = jax.nn.relu(jnp.dot(x_ref[...], w1t_ref[...], preferred_element_type=F32)
                    + b1_ref[...])
    y = jnp.dot(h, w2t_ref[...], preferred_element_type=F32) + b2_ref[...]
    o_ref[...] = 1.0 / (1.0 + jnp.exp(-y))


def _full_spec(shape):
    return pl.BlockSpec(shape, lambda i: tuple(0 for _ in shape))


def kernel(constraint_features, edge_index, edge_attr, variable_features, batch, params):
    p = params
    nvars, fvar = variable_features.shape
    ncons = constraint_features.shape[0]
    nedges = edge_index.shape[1]
    del constraint_features, edge_attr

    ef = p['ee_ln_b'][0]
    cpre = p['ce_ln_b'][0]
    x1 = jax.nn.relu(cpre * p['ce_W1'][:, 0] + p['ce_b1'])
    cv = jax.nn.relu(p['ce_W2'] @ x1 + p['ce_b2'])

    convs = p['convs']
    stk = lambda f: jnp.stack([f(c) for c in convs])
    wrt = stk(lambda c: c['Wr'].T)
    base = stk(lambda c: (c['Wl'] @ cv + c['bl'] + ef * c['We'][:, 0])[None, :])
    fg = stk(lambda c: c['fin_ln_g'][None, :])
    fb = stk(lambda c: c['fin_ln_b'][None, :])
    wft = stk(lambda c: c['Wf'].T)
    bf = stk(lambda c: c['bf'][None, :])
    pg = stk(lambda c: c['post_ln_g'][None, :])
    pb = stk(lambda c: c['post_ln_b'][None, :])
    at_ = stk(lambda c: c['Wo1'][:, :16].T)
    c2 = stk(lambda c: (c['Wo1'][:, 16:] @ cv + c['bo1'])[None, :])
    wo2t = stk(lambda c: c['Wo2'].T)
    bo2 = stk(lambda c: c['bo2'][None, :])

    RA = 1000
    u4 = pl.pallas_call(
        _var_table_body,
        grid=(nvars // RA,),
        in_specs=[pl.BlockSpec((RA, fvar), lambda i: (i, 0)),
                  _full_spec((1, fvar)), _full_spec((1, fvar)),
                  _full_spec((fvar, 16)), _full_spec((1, 16)),
                  _full_spec((16, 16)), _full_spec((1, 16)),
                  _full_spec((NCONV, 16, 16)), _full_spec((NCONV, 1, 16)),
                  _full_spec((NCONV, 1, 16)), _full_spec((NCONV, 1, 16)),
                  _full_spec((NCONV, 16, 16)), _full_spec((NCONV, 1, 16))],
        out_specs=pl.BlockSpec((NCONV, RA, 16), lambda i: (0, i, 0)),
        out_shape=jax.ShapeDtypeStruct((NCONV, nvars, 16), F32),
    )(variable_features,
      p['ve_ln_g'][None, :], p['ve_ln_b'][None, :],
      p['ve_W1'].T, p['ve_b1'][None, :], p['ve_W2'].T, p['ve_b2'][None, :],
      wrt, base, fg, fb, wft, bf)

    ncons_p = 100096
    stripe = ncons_p // NSUB
    BLK = 4
    cpw = -(-nedges // (NSUB * 128))
    cps = -(-cpw // BLK) * BLK
    nblk = cps // BLK
    epad = NSUB * cps * 128
    srcp = jnp.concatenate(
        [edge_index[0], jnp.zeros((epad - nedges,), jnp.int32)]
    ).reshape(NSUB, nblk, BLK, 128)
    dstp = jnp.concatenate(
        [edge_index[1], jnp.full((epad - nedges,), ncons, jnp.int32)]
    ).reshape(NSUB, nblk, BLK, 128)
    zeros_hbm = jnp.zeros((stripe, 16), F32)

    mesh = plsc.VectorSubcoreMesh(core_axis_name="c", subcore_axis_name="s",
                                  num_cores=NSC, num_subcores=NSUB)

    @functools.partial(
        pl.kernel, mesh=mesh,
        compiler_params=pltpu.CompilerParams(use_tc_tiling_on_sc=False, needs_layout_passes=False),
        out_type=jax.ShapeDtypeStruct((NCONV, ncons_p, 16), F32),
        scratch_types=[
            pltpu.VMEM((2, BLK, 128), jnp.int32),
            pltpu.VMEM((2, BLK, 128), jnp.int32),
            pltpu.VMEM((2, BLK, 128, 16), F32),
            pltpu.VMEM_SHARED((ncons_p, 16), F32),
            pltpu.SemaphoreType.DMA,
            pltpu.SemaphoreType.DMA,
            pltpu.SemaphoreType.DMA,
        ])
    def _edge_kernel(u_hbm, src_hbm, dst_hbm, z_hbm, out_hbm,
                     sblk, dblk, rows, acc, gsem, ssem, isem):
        core = lax.axis_index("c")
        sid = lax.axis_index("s")

        def fire_gathers(utab, q):
            for j in range(BLK):
                pltpu.async_copy(utab.at[sblk.at[q].at[j]], rows.at[q].at[j], gsem)

        def drain_gathers(utab, q):
            for j in range(BLK):
                pltpu.make_async_copy(utab.at[sblk.at[q].at[j]],
                                      rows.at[q].at[j], gsem).wait()

        def fire_scatters(q):
            for j in range(BLK):
                pltpu.async_copy(rows.at[q].at[j], acc.at[dblk.at[q].at[j]],
                                 ssem, add=True)

        def drain_scatters(q):
            for j in range(BLK):
                pltpu.make_async_copy(rows.at[q].at[j],
                                      acc.at[dblk.at[q].at[j]], ssem).wait()

        for pss in range(2):
            conv = core * 2 + pss
            pltpu.sync_copy(z_hbm, acc.at[pl.ds(sid * stripe, stripe)])
            plsc.subcore_barrier()
            utab = u_hbm.at[conv]

            pltpu.sync_copy(src_hbm.at[sid, 0], sblk.at[0])
            pltpu.sync_copy(dst_hbm.at[sid, 0], dblk.at[0])
            fire_gathers(utab, 0)

            def blk_body(b, _):
                q = lax.rem(b, 2)
                nq = 1 - q

                @pl.when(b > 0)
                def _():
                    drain_scatters(nq)

                @pl.when(b + 1 < nblk)
                def _():
                    pltpu.async_copy(src_hbm.at[sid, b + 1], sblk.at[nq], isem)
                    pltpu.async_copy(dst_hbm.at[sid, b + 1], dblk.at[nq], isem)

                drain_gathers(utab, q)
                fire_scatters(q)

                @pl.when(b + 1 < nblk)
                def _():
                    pltpu.make_async_copy(src_hbm.at[sid, b + 1],
                                          sblk.at[nq], isem).wait()
                    pltpu.make_async_copy(dst_hbm.at[sid, b + 1],
                                          dblk.at[nq], isem).wait()
                    fire_gathers(utab, nq)

                return 0

            lax.fori_loop(0, nblk, blk_body, 0)
            drain_scatters(lax.rem(nblk - 1, 2))
            plsc.subcore_barrier()
            pltpu.sync_copy(acc.at[pl.ds(sid * stripe, stripe)],
                            out_hbm.at[conv].at[pl.ds(sid * stripe, stripe)])
            plsc.subcore_barrier()

    aggr4 = _edge_kernel(u4, srcp, dstp, zeros_hbm)

    RC = 800
    nh = 100128
    h4 = pl.pallas_call(
        _post_body,
        grid=(ncons // RC,),
        in_specs=[pl.BlockSpec((NCONV, RC, 16), lambda i: (0, i, 0)),
                  _full_spec((NCONV, 1, 16)), _full_spec((NCONV, 1, 16)),
                  _full_spec((NCONV, 16, 16)), _full_spec((NCONV, 1, 16)),
                  _full_spec((NCONV, 16, 16)), _full_spec((NCONV, 1, 16))],
        out_specs=pl.BlockSpec((NCONV, RC, 16), lambda i: (0, i, 0)),
        out_shape=jax.ShapeDtypeStruct((NCONV, nh, 16), F32),
    )(aggr4, pg, pb, at_, c2, wo2t, bo2)
    hrows = h4.reshape(NCONV * nh, 16)

    gr = jnp.arange(NUM_GRAPHS, dtype=batch.dtype)
    starts = jnp.searchsorted(batch, gr, side='left').astype(jnp.int32)
    ends = jnp.searchsorted(batch, gr, side='right').astype(jnp.int32)
    item_s = jnp.repeat(starts, NCONV)
    item_e = jnp.repeat(ends, NCONV)

    nitems = NUM_GRAPHS * NCONV
    per_sub = nitems // (NSC * NSUB)

    @functools.partial(
        pl.kernel, mesh=mesh,
        compiler_params=pltpu.CompilerParams(use_tc_tiling_on_sc=False, needs_layout_passes=False),
        out_type=jax.ShapeDtypeStruct((nitems * KTOP, 16), F32),
        scratch_types=[
            pltpu.VMEM((128, 16), F32),
            pltpu.VMEM((per_sub,), jnp.int32),
            pltpu.VMEM((per_sub,), jnp.int32),
            pltpu.VMEM((16,), jnp.int32),
            pltpu.VMEM((16, 16), F32),
            pltpu.SemaphoreType.DMA,
        ])
    def _pool_kernel(hr_hbm, its_hbm, ite_hbm, out_hbm,
                     stage, msv, mev, idxb, rows16, sem):
        core = lax.axis_index("c")
        sid = lax.axis_index("s")
        wid = sid * NSC + core
        pltpu.sync_copy(its_hbm.at[pl.ds(wid * per_sub, per_sub)], msv)
        pltpu.sync_copy(ite_hbm.at[pl.ds(wid * per_sub, per_sub)], mev)
        iot = lax.iota(jnp.int32, 16)
        NEG = jnp.float32(-jnp.inf)
        SENT = jnp.int32(2**31 - 1)
        lane15 = jnp.full((16,), 15, jnp.int32)

        def item_body(i, _):
            it = wid * per_sub + i
            cix = lax.rem(it, NCONV)
            sel = iot == i
            s = jnp.sum(jnp.where(sel, msv[...], 0))
            e = jnp.sum(jnp.where(sel, mev[...], 0))
            base0 = cix * nh
            s8 = jnp.bitwise_and(s, jnp.int32(-8))
            nst = (e - s8 + 127) // 128

            def stage_body(t, carry):
                runk, runi = carry
                r0 = base0 + s8 + t * 128
                pltpu.sync_copy(hr_hbm.at[pl.ds(r0, 128)], stage)
                for q in range(8):
                    kq = plsc.load_gather(stage, [q * 16 + iot, lane15])
                    ids = s8 + t * 128 + q * 16 + iot
                    valid = (ids >= s) & (ids < e)
                    kq = jnp.where(valid, kq, NEG)
                    idq = jnp.where(valid, ids, SENT)
                    sk, si = plsc.sort_key_val(kq, idq, descending=True)
                    rb = lax.rev(sk, (0,))
                    ri = lax.rev(si, (0,))
                    take = (runk > rb) | ((runk == rb) & (runi <= ri))
                    mk = jnp.where(take, runk, rb)
                    mi = jnp.where(take, runi, ri)
                    runk, runi = plsc.sort_key_val(mk, mi, descending=True)
                return runk, runi

            runk, runi = lax.fori_loop(
                0, nst, stage_body,
                (jnp.full((16,), NEG), jnp.full((16,), SENT)))
            valid16 = runi != SENT
            idxb[...] = base0 + jnp.where(valid16, runi, 0)
            pltpu.async_copy(hr_hbm.at[idxb], rows16, sem).wait()
            for r in range(16):
                vr = jnp.sum(jnp.where((iot == r) & valid16, 1, 0)) > 0
                rows16[r, :] = jnp.where(vr, rows16[r, :], jnp.zeros((16,), F32))
            pltpu.sync_copy(rows16, out_hbm.at[pl.ds(it * KTOP, KTOP)])
            return 0

        lax.fori_loop(0, per_sub, item_body, 0)

    pooled = _pool_kernel(hrows, item_s, item_e)

    cat = pooled.reshape(NUM_GRAPHS, NCONV * KTOP * 16)
    out = pl.pallas_call(
        _mlp_body,
        out_shape=jax.ShapeDtypeStruct((NUM_GRAPHS, 1), F32),
    )(cat, p['m_W1'].T, p['m_b1'][None, :], p['m_W2'].T, p['m_b2'][None, :])
    return out[:, 0]
